# baseline (speedup 1.0000x reference)
import functools

import jax
import jax.numpy as jnp
from jax import lax
from jax.experimental import pallas as pl
from jax.experimental.pallas import tpu as pltpu
from jax.experimental.pallas import tpu_sc as plsc

N_CLUSTERS = 100000
HIDDEN = 64
SEQ = 200

NUM_CORES = 2
NUM_SUBCORES = 16
NW = NUM_CORES * NUM_SUBCORES

CHUNK = SEQ
GSLICE = 40
NGS = CHUNK // GSLICE
TOTAL = 4096 * SEQ
NCHUNKS = TOTAL // CHUNK
CH_PER_W = NCHUNKS // NW
NBUF = 4
LOOK = 2


def _body(x_hbm, pix_hbm, pos_hbm, out_hbm,
          idx_all, rows0, rows1, rows2, rows3, pos_v,
          sem_g0, sem_g1, sem_g2, sem_g3,
          sem_o0, sem_o1, sem_o2, sem_o3):
    wid = lax.axis_index("s") * NUM_CORES + lax.axis_index("c")
    base = wid * CH_PER_W
    rows = (rows0, rows1, rows2, rows3)
    sem_g = (sem_g0, sem_g1, sem_g2, sem_g3)
    sem_o = (sem_o0, sem_o1, sem_o2, sem_o3)

    pltpu.sync_copy(pos_hbm.at[pl.ds(0, SEQ)], pos_v)
    pltpu.sync_copy(x_hbm.at[wid], idx_all)

    def fire_gathers(c, b):
        for k in range(NGS):
            pltpu.async_copy(
                pix_hbm.at[idx_all.at[c, k]],
                rows[b].at[pl.ds(k * GSLICE, GSLICE)],
                sem_g[b],
            )

    def fire_out(c, b):
        pltpu.async_copy(rows[b], out_hbm.at[base + c], sem_o[b])

    def drain_out(c, b):
        pltpu.make_async_copy(rows[b], out_hbm.at[base + c], sem_o[b]).wait()

    def _chunk_step(c, b, bn):
        @pl.when(c + LOOK < CH_PER_W)
        def _fire_ahead():
            @pl.when(c + LOOK >= NBUF)
            def _drain_prev_out():
                drain_out(c + LOOK - NBUF, bn)
            fire_gathers(c + LOOK, bn)

        pltpu.make_async_copy(
            pix_hbm.at[pl.ds(0, CHUNK)], rows[b], sem_g[b]
        ).wait()

        @pl.loop(0, SEQ, unroll=8)
        def _add(r):
            for j in range(HIDDEN // 16):
                sl = pl.ds(j * 16, 16)
                plsc.addupdate(rows[b].at[r, sl], pos_v[r, sl])

        fire_out(c, b)

    for p in range(LOOK):
        fire_gathers(p, p)

    @pl.loop(0, CH_PER_W, step=NBUF)
    def _chunk_grp(c0):
        for b in range(NBUF):
            _chunk_step(c0 + b, b, (b + LOOK) % NBUF)

    for q in range(NBUF):
        c = CH_PER_W - NBUF + q
        drain_out(c, c % NBUF)


def kernel(x, pix_table, pos_table):
    b, seq = x.shape
    x4 = x.astype(jnp.int32).reshape(NW, CH_PER_W, NGS, GSLICE)
    mesh = plsc.VectorSubcoreMesh(core_axis_name="c", subcore_axis_name="s")
    run = functools.partial(
        pl.kernel,
        mesh=mesh,
        out_type=jax.ShapeDtypeStruct((4096, SEQ, HIDDEN), jnp.float32),
        scratch_types=[
            pltpu.VMEM((CH_PER_W, NGS, GSLICE), jnp.int32),
            pltpu.VMEM((CHUNK, HIDDEN), jnp.float32),
            pltpu.VMEM((CHUNK, HIDDEN), jnp.float32),
            pltpu.VMEM((CHUNK, HIDDEN), jnp.float32),
            pltpu.VMEM((CHUNK, HIDDEN), jnp.float32),
            pltpu.VMEM((SEQ, HIDDEN), jnp.float32),
            pltpu.SemaphoreType.DMA,
            pltpu.SemaphoreType.DMA,
            pltpu.SemaphoreType.DMA,
            pltpu.SemaphoreType.DMA,
            pltpu.SemaphoreType.DMA,
            pltpu.SemaphoreType.DMA,
            pltpu.SemaphoreType.DMA,
            pltpu.SemaphoreType.DMA,
        ],
        compiler_params=pltpu.CompilerParams(use_tc_tiling_on_sc=False),
    )(_body)
    return run(x4, pix_table, pos_table)


# speedup vs baseline: 1.0175x; 1.0175x over previous
import functools

import jax
import jax.numpy as jnp
from jax import lax
from jax.experimental import pallas as pl
from jax.experimental.pallas import tpu as pltpu
from jax.experimental.pallas import tpu_sc as plsc

N_CLUSTERS = 100000
HIDDEN = 64
SEQ = 200

NUM_CORES = 2
NUM_SUBCORES = 16
NW = NUM_CORES * NUM_SUBCORES

CHUNK = 2 * SEQ
SPLITS = (104, 96)
TOTAL = 4096 * SEQ
NCHUNKS = TOTAL // CHUNK
CH_PER_W = NCHUNKS // NW
NBUF = 3
LOOK = 1


def _body(x_hbm, pix_hbm, pos_hbm, out_hbm,
          idx_all, rows0, rows1, rows2, pos_v,
          sem_a0, sem_a1, sem_a2,
          sem_b0, sem_b1, sem_b2,
          sem_o0, sem_o1, sem_o2):
    wid = lax.axis_index("s") * NUM_CORES + lax.axis_index("c")
    base2 = wid * (2 * CH_PER_W)
    rows = (rows0, rows1, rows2)
    sem_a = (sem_a0, sem_a1, sem_a2)
    sem_b = (sem_b0, sem_b1, sem_b2)
    sem_o = (sem_o0, sem_o1, sem_o2)

    pltpu.sync_copy(pos_hbm.at[pl.ds(0, SEQ)], pos_v)
    pltpu.sync_copy(x_hbm.at[wid], idx_all)

    def fire_gathers(c, b):
        for half, sem in ((0, sem_a), (1, sem_b)):
            off = half * SEQ
            for L in SPLITS:
                pltpu.async_copy(
                    pix_hbm.at[idx_all.at[c, pl.ds(off, L)]],
                    rows[b].at[pl.ds(off, L)],
                    sem[b],
                )
                off += L

    def drain_half(b, half, sem):
        pltpu.make_async_copy(
            pix_hbm.at[pl.ds(0, SEQ)],
            rows[b].at[pl.ds(half * SEQ, SEQ)],
            sem[b],
        ).wait()

    def fire_out_half(c, b, half):
        pltpu.async_copy(
            rows[b].at[pl.ds(half * SEQ, SEQ)],
            out_hbm.at[base2 + 2 * c + half],
            sem_o[b],
        )

    def drain_out(c, b):
        for half in range(2):
            pltpu.make_async_copy(
                rows[b].at[pl.ds(half * SEQ, SEQ)],
                out_hbm.at[base2 + 2 * c + half],
                sem_o[b],
            ).wait()

    def add_half(b, half):
        @pl.loop(0, SEQ, unroll=8)
        def _add(r):
            for j in range(HIDDEN // 16):
                sl = pl.ds(j * 16, 16)
                plsc.addupdate(rows[b].at[half * SEQ + r, sl], pos_v[r, sl])

    def _chunk_step(c, b, bn):
        @pl.when(c + LOOK < CH_PER_W)
        def _fire_ahead():
            @pl.when(c + LOOK >= NBUF)
            def _drain_prev_out():
                drain_out(c + LOOK - NBUF, bn)
            fire_gathers(c + LOOK, bn)

        drain_half(b, 0, sem_a)
        add_half(b, 0)
        fire_out_half(c, b, 0)

        drain_half(b, 1, sem_b)
        add_half(b, 1)
        fire_out_half(c, b, 1)

    for p in range(LOOK):
        fire_gathers(p, p)

    @pl.loop(0, CH_PER_W, step=NBUF)
    def _chunk_grp(c0):
        for b in range(NBUF):
            c = c0 + b
            bn = (b + LOOK) % NBUF

            @pl.when(c < CH_PER_W)
            def _in_range():
                _chunk_step(c, b, bn)

    for q in range(NBUF):
        c = CH_PER_W - NBUF + q
        drain_out(c, c % NBUF)


def kernel(x, pix_table, pos_table):
    b, seq = x.shape
    x4 = x.astype(jnp.int32).reshape(NW, CH_PER_W, CHUNK)
    mesh = plsc.VectorSubcoreMesh(core_axis_name="c", subcore_axis_name="s")
    run = functools.partial(
        pl.kernel,
        mesh=mesh,
        out_type=jax.ShapeDtypeStruct((4096, SEQ, HIDDEN), jnp.float32),
        scratch_types=[
            pltpu.VMEM((CH_PER_W, CHUNK), jnp.int32),
            pltpu.VMEM((CHUNK, HIDDEN), jnp.float32),
            pltpu.VMEM((CHUNK, HIDDEN), jnp.float32),
            pltpu.VMEM((CHUNK, HIDDEN), jnp.float32),
            pltpu.VMEM((SEQ, HIDDEN), jnp.float32),
            pltpu.SemaphoreType.DMA,
            pltpu.SemaphoreType.DMA,
            pltpu.SemaphoreType.DMA,
            pltpu.SemaphoreType.DMA,
            pltpu.SemaphoreType.DMA,
            pltpu.SemaphoreType.DMA,
            pltpu.SemaphoreType.DMA,
            pltpu.SemaphoreType.DMA,
            pltpu.SemaphoreType.DMA,
        ],
        compiler_params=pltpu.CompilerParams(use_tc_tiling_on_sc=False),
    )(_body)
    return run(x4, pix_table, pos_table)
